# Initial kernel scaffold; baseline (speedup 1.0000x reference)
import jax
import jax.numpy as jnp
from jax.experimental import pallas as pl


def kernel(x, edge_index, batch, W1, b1, gamma, beta, run_mean, run_var, W2, b2):
    raise NotImplementedError("write your pallas kernel here")


# speedup vs baseline: 14.6400x; 14.6400x over previous
import functools

import jax
import jax.numpy as jnp
from jax import lax
from jax.experimental import pallas as pl
from jax.experimental.pallas import tpu as pltpu
from jax.experimental.pallas import tpu_sc as plsc

N = 10000
E = 320000
D = 128
DH = D // 2
EPS = 1e-5

NC = 2
NS = 16
NW = NC * NS
CK = 80
EPW = E // NW
NIT = EPW // CK
EPT = E // NS
NI2 = EPT // CK
NP = 10240
RPT = NP // NS
ZR = 160
RB = 1024

_mesh = plsc.VectorSubcoreMesh(core_axis_name="c", subcore_axis_name="s")


@functools.partial(
    pl.kernel,
    mesh=_mesh,
    out_type=jax.ShapeDtypeStruct((NC, NP, 16), jnp.float32),
    scratch_types=[
        pltpu.VMEM((NIT, CK), jnp.int32),
        pltpu.VMEM((CK, 16), jnp.float32),
        pltpu.VMEM((RPT, 16), jnp.float32),
        pltpu.VMEM_SHARED((NP, 16), jnp.float32),
        pltpu.SemaphoreType.DMA,
    ],
    compiler_params=pltpu.CompilerParams(use_tc_tiling_on_sc=False),
)
def _deg_kernel(dst_hbm, out_hbm, idx_v, ones_v, buf_v, acc_sh, sem):
    c = lax.axis_index("c")
    s = lax.axis_index("s")
    wid = s * NC + c

    def fill(i, _):
        ones_v[i] = jnp.full((16,), 1.0, jnp.float32)
        return 0

    lax.fori_loop(0, CK, fill, 0)

    def fill2(i, _):
        buf_v[i] = jnp.zeros((16,), jnp.float32)
        return 0

    lax.fori_loop(0, RPT, fill2, 0)

    pltpu.sync_copy(buf_v, acc_sh.at[pl.ds(s * RPT, RPT)])
    plsc.subcore_barrier()

    pltpu.sync_copy(dst_hbm.at[wid], idx_v)

    def body(j, _):
        pltpu.sync_copy(ones_v, acc_sh.at[idx_v.at[j]], add=True)
        return 0

    lax.fori_loop(0, NIT, body, 0)
    plsc.subcore_barrier()

    pltpu.sync_copy(acc_sh.at[pl.ds(s * RPT, RPT)], buf_v)
    pltpu.sync_copy(buf_v, out_hbm.at[c, pl.ds(s * RPT, RPT)])


@functools.partial(
    pl.kernel,
    mesh=_mesh,
    out_type=jax.ShapeDtypeStruct((NC, NP, DH), jnp.float32),
    scratch_types=[
        pltpu.VMEM((NI2, CK), jnp.int32),
        pltpu.VMEM((NI2, CK), jnp.int32),
        pltpu.VMEM((CK, DH), jnp.float32),
        pltpu.VMEM((ZR, DH), jnp.float32),
        pltpu.VMEM_SHARED((NP, DH), jnp.float32),
        pltpu.SemaphoreType.DMA,
    ],
    compiler_params=pltpu.CompilerParams(use_tc_tiling_on_sc=False),
)
def _scatter_kernel(src_hbm, dst_hbm, g_hbm, out_hbm, srcv, dstv, rows_v,
                    buf_v, acc_sh, sem):
    c = lax.axis_index("c")
    s = lax.axis_index("s")

    def fill(i, _):
        for k in range(DH // 16):
            buf_v[i, pl.ds(16 * k, 16)] = jnp.zeros((16,), jnp.float32)
        return 0

    lax.fori_loop(0, ZR, fill, 0)

    for t in range(RPT // ZR):
        pltpu.sync_copy(buf_v, acc_sh.at[pl.ds(s * RPT + t * ZR, ZR)])
    plsc.subcore_barrier()

    pltpu.sync_copy(src_hbm.at[s], srcv)
    pltpu.sync_copy(dst_hbm.at[s], dstv)

    off = c * NP

    def adjust(j, _):
        for k in range(CK // 16):
            sl = pl.ds(16 * k, 16)
            srcv[j, sl] = srcv[j, sl] + off
        return 0

    lax.fori_loop(0, NI2, adjust, 0)

    def body(j, _):
        pltpu.async_copy(g_hbm.at[srcv.at[j]], rows_v, sem).wait()
        pltpu.sync_copy(rows_v, acc_sh.at[dstv.at[j]], add=True)
        return 0

    lax.fori_loop(0, NI2, body, 0)
    plsc.subcore_barrier()

    for t in range(RPT // ZR):
        sl = pl.ds(s * RPT + t * ZR, ZR)
        pltpu.sync_copy(acc_sh.at[sl], buf_v)
        pltpu.sync_copy(buf_v, out_hbm.at[c, sl])


def _tca_body(degp_ref, x_ref, w_ref, dinv_ref, g_ref):
    deg = degp_ref[0, :, 0:1] + degp_ref[1, :, 0:1] + 1.0
    dinv = lax.rsqrt(deg)
    h = jnp.dot(x_ref[...], w_ref[...], preferred_element_type=jnp.float32)
    g = h * dinv
    dinv_ref[...] = dinv
    g_ref[0] = g[:, :DH]
    g_ref[1] = g[:, DH:]


_tca = pl.pallas_call(
    _tca_body,
    grid=(NP // RB,),
    in_specs=[
        pl.BlockSpec((2, RB, 16), lambda i: (0, i, 0)),
        pl.BlockSpec((RB, D), lambda i: (i, 0)),
        pl.BlockSpec((D, D), lambda i: (0, 0)),
    ],
    out_specs=[
        pl.BlockSpec((RB, 1), lambda i: (i, 0)),
        pl.BlockSpec((2, RB, DH), lambda i: (0, i, 0)),
    ],
    out_shape=[
        jax.ShapeDtypeStruct((NP, 1), jnp.float32),
        jax.ShapeDtypeStruct((2, NP, DH), jnp.float32),
    ],
)


def _tcb_body(sp_ref, g1_ref, dinv_ref, b1_ref, a1_ref, c1_ref, w2_ref,
              g2_ref):
    sval = jnp.concatenate(
        [sp_ref[0] + g1_ref[0], sp_ref[1] + g1_ref[1]], axis=-1)
    dinv = dinv_ref[...]
    o = jnp.maximum(sval * dinv + b1_ref[...], 0.0)
    h = o * a1_ref[...] + c1_ref[...]
    hw = jnp.dot(h, w2_ref[...], preferred_element_type=jnp.float32)
    g2 = hw * dinv
    g2_ref[0] = g2[:, :DH]
    g2_ref[1] = g2[:, DH:]


_tcb = pl.pallas_call(
    _tcb_body,
    grid=(NP // RB,),
    in_specs=[
        pl.BlockSpec((2, RB, DH), lambda i: (0, i, 0)),
        pl.BlockSpec((2, RB, DH), lambda i: (0, i, 0)),
        pl.BlockSpec((RB, 1), lambda i: (i, 0)),
        pl.BlockSpec((1, D), lambda i: (0, 0)),
        pl.BlockSpec((1, D), lambda i: (0, 0)),
        pl.BlockSpec((1, D), lambda i: (0, 0)),
        pl.BlockSpec((D, D), lambda i: (0, 0)),
    ],
    out_specs=pl.BlockSpec((2, RB, DH), lambda i: (0, i, 0)),
    out_shape=jax.ShapeDtypeStruct((2, NP, DH), jnp.float32),
)


def _tcc_body(sp_ref, g2_ref, dinv_ref, b2_ref, y_ref):
    sval = jnp.concatenate(
        [sp_ref[0] + g2_ref[0], sp_ref[1] + g2_ref[1]], axis=-1)
    y_ref[...] = jnp.maximum(sval * dinv_ref[...] + b2_ref[...], 0.0)


_tcc = pl.pallas_call(
    _tcc_body,
    grid=(NP // RB,),
    in_specs=[
        pl.BlockSpec((2, RB, DH), lambda i: (0, i, 0)),
        pl.BlockSpec((2, RB, DH), lambda i: (0, i, 0)),
        pl.BlockSpec((RB, 1), lambda i: (i, 0)),
        pl.BlockSpec((1, D), lambda i: (0, 0)),
    ],
    out_specs=pl.BlockSpec((RB, D), lambda i: (i, 0)),
    out_shape=jax.ShapeDtypeStruct((NP, D), jnp.float32),
)


def kernel(x, edge_index, batch, W1, b1, gamma, beta, run_mean, run_var,
           W2, b2):
    src16 = edge_index[0].reshape(NS, NI2, CK)
    dst16 = edge_index[1].reshape(NS, NI2, CK)
    dst32 = edge_index[1].reshape(NW, NIT, CK)

    degp = _deg_kernel(dst32)

    x_pad = jnp.pad(x, ((0, NP - N), (0, 0)))
    dinv, g1 = _tca(degp, x_pad, W1)

    s1 = _scatter_kernel(src16, dst16, g1.reshape(2 * NP, DH))

    a1 = gamma * lax.rsqrt(run_var + EPS)
    c1 = beta - run_mean * a1
    g2 = _tcb(s1, g1, dinv, b1[None, :], a1[None, :], c1[None, :], W2)

    s2 = _scatter_kernel(src16, dst16, g2.reshape(2 * NP, DH))
    y = _tcc(s2, g2, dinv, b2[None, :])
    return y[:N]


# speedup vs baseline: 31.7300x; 2.1673x over previous
import functools

import jax
import jax.numpy as jnp
from jax import lax
from jax.experimental import pallas as pl
from jax.experimental.pallas import tpu as pltpu
from jax.experimental.pallas import tpu_sc as plsc

N = 10000
E = 320000
D = 128
DH = D // 2
EPS = 1e-5

NC = 2
NS = 16
NW = NC * NS
CK = 80
EPW = E // NW
NIT = EPW // CK
EPT = E // NS
NI2 = EPT // CK
NP = 10240
RPT = NP // NS
ZR = 160
RB = 1024
NBUF = 5

_mesh = plsc.VectorSubcoreMesh(core_axis_name="c", subcore_axis_name="s")


@functools.partial(
    pl.kernel,
    mesh=_mesh,
    out_type=jax.ShapeDtypeStruct((NC, NP, 16), jnp.float32),
    scratch_types=[
        pltpu.VMEM((NIT, CK), jnp.int32),
        pltpu.VMEM((CK, 16), jnp.float32),
        pltpu.VMEM((RPT, 16), jnp.float32),
        pltpu.VMEM_SHARED((NP, 16), jnp.float32),
        pltpu.SemaphoreType.DMA,
    ],
    compiler_params=pltpu.CompilerParams(use_tc_tiling_on_sc=False),
)
def _deg_kernel(dst_hbm, out_hbm, idx_v, ones_v, buf_v, acc_sh, sem):
    c = lax.axis_index("c")
    s = lax.axis_index("s")
    wid = s * NC + c

    def fill(i, _):
        ones_v[i] = jnp.full((16,), 1.0, jnp.float32)
        return 0

    lax.fori_loop(0, CK, fill, 0)

    def fill2(i, _):
        buf_v[i] = jnp.zeros((16,), jnp.float32)
        return 0

    lax.fori_loop(0, RPT, fill2, 0)

    pltpu.sync_copy(buf_v, acc_sh.at[pl.ds(s * RPT, RPT)])
    plsc.subcore_barrier()

    pltpu.sync_copy(dst_hbm.at[wid], idx_v)

    def body(j, _):
        pltpu.sync_copy(ones_v, acc_sh.at[idx_v.at[j]], add=True)
        return 0

    lax.fori_loop(0, NIT, body, 0)
    plsc.subcore_barrier()

    pltpu.sync_copy(acc_sh.at[pl.ds(s * RPT, RPT)], buf_v)
    pltpu.sync_copy(buf_v, out_hbm.at[c, pl.ds(s * RPT, RPT)])


@functools.partial(
    pl.kernel,
    mesh=_mesh,
    out_type=jax.ShapeDtypeStruct((NC, NP, DH), jnp.float32),
    scratch_types=[
        pltpu.VMEM((NI2, CK), jnp.int32),
        pltpu.VMEM((NI2, CK), jnp.int32),
        pltpu.VMEM((NBUF, CK, DH), jnp.float32),
        pltpu.VMEM((ZR, DH), jnp.float32),
        pltpu.VMEM_SHARED((NP, DH), jnp.float32),
    ] + [pltpu.SemaphoreType.DMA] * NBUF,
    compiler_params=pltpu.CompilerParams(use_tc_tiling_on_sc=False),
)
def _scatter_kernel(src_hbm, dst_hbm, g_hbm, out_hbm, srcv, dstv, rows_v,
                    buf_v, acc_sh, *sems):
    c = lax.axis_index("c")
    s = lax.axis_index("s")

    def fill(i, _):
        for k in range(DH // 16):
            buf_v[i, pl.ds(16 * k, 16)] = jnp.zeros((16,), jnp.float32)
        return 0

    lax.fori_loop(0, ZR, fill, 0)

    for t in range(RPT // ZR):
        pltpu.sync_copy(buf_v, acc_sh.at[pl.ds(s * RPT + t * ZR, ZR)])
    plsc.subcore_barrier()

    pltpu.sync_copy(src_hbm.at[s], srcv)
    pltpu.sync_copy(dst_hbm.at[s], dstv)

    off = c * NP

    def adjust(j, _):
        for k in range(CK // 16):
            sl = pl.ds(16 * k, 16)
            srcv[j, sl] = srcv[j, sl] + off
        return 0

    lax.fori_loop(0, NI2, adjust, 0)

    for b in range(NBUF):
        pltpu.async_copy(g_hbm.at[srcv.at[b]], rows_v.at[b], sems[b])

    def grp(g, _):
        for b in range(NBUF):
            j = g * NBUF + b
            pltpu.make_async_copy(
                g_hbm.at[srcv.at[j]], rows_v.at[b], sems[b]).wait()
            pltpu.sync_copy(rows_v.at[b], acc_sh.at[dstv.at[j]], add=True)
            nj = j + NBUF

            @pl.when(nj < NI2)
            def _():
                pltpu.async_copy(g_hbm.at[srcv.at[nj]], rows_v.at[b],
                                 sems[b])
        return 0

    lax.fori_loop(0, NI2 // NBUF, grp, 0)
    plsc.subcore_barrier()

    for t in range(RPT // ZR):
        sl = pl.ds(s * RPT + t * ZR, ZR)
        pltpu.sync_copy(acc_sh.at[sl], buf_v)
        pltpu.sync_copy(buf_v, out_hbm.at[c, sl])


def _tca_body(degp_ref, x_ref, w_ref, dinv_ref, g_ref):
    deg = degp_ref[0, :, 0:1] + degp_ref[1, :, 0:1] + 1.0
    dinv = lax.rsqrt(deg)
    h = jnp.dot(x_ref[...], w_ref[...], preferred_element_type=jnp.float32)
    g = h * dinv
    dinv_ref[...] = dinv
    g_ref[0] = g[:, :DH]
    g_ref[1] = g[:, DH:]


_tca = pl.pallas_call(
    _tca_body,
    grid=(NP // RB,),
    in_specs=[
        pl.BlockSpec((2, RB, 16), lambda i: (0, i, 0)),
        pl.BlockSpec((RB, D), lambda i: (i, 0)),
        pl.BlockSpec((D, D), lambda i: (0, 0)),
    ],
    out_specs=[
        pl.BlockSpec((RB, 1), lambda i: (i, 0)),
        pl.BlockSpec((2, RB, DH), lambda i: (0, i, 0)),
    ],
    out_shape=[
        jax.ShapeDtypeStruct((NP, 1), jnp.float32),
        jax.ShapeDtypeStruct((2, NP, DH), jnp.float32),
    ],
)


def _tcb_body(sp_ref, g1_ref, dinv_ref, b1_ref, a1_ref, c1_ref, w2_ref,
              g2_ref):
    sval = jnp.concatenate(
        [sp_ref[0] + g1_ref[0], sp_ref[1] + g1_ref[1]], axis=-1)
    dinv = dinv_ref[...]
    o = jnp.maximum(sval * dinv + b1_ref[...], 0.0)
    h = o * a1_ref[...] + c1_ref[...]
    hw = jnp.dot(h, w2_ref[...], preferred_element_type=jnp.float32)
    g2 = hw * dinv
    g2_ref[0] = g2[:, :DH]
    g2_ref[1] = g2[:, DH:]


_tcb = pl.pallas_call(
    _tcb_body,
    grid=(NP // RB,),
    in_specs=[
        pl.BlockSpec((2, RB, DH), lambda i: (0, i, 0)),
        pl.BlockSpec((2, RB, DH), lambda i: (0, i, 0)),
        pl.BlockSpec((RB, 1), lambda i: (i, 0)),
        pl.BlockSpec((1, D), lambda i: (0, 0)),
        pl.BlockSpec((1, D), lambda i: (0, 0)),
        pl.BlockSpec((1, D), lambda i: (0, 0)),
        pl.BlockSpec((D, D), lambda i: (0, 0)),
    ],
    out_specs=pl.BlockSpec((2, RB, DH), lambda i: (0, i, 0)),
    out_shape=jax.ShapeDtypeStruct((2, NP, DH), jnp.float32),
)


def _tcc_body(sp_ref, g2_ref, dinv_ref, b2_ref, y_ref):
    sval = jnp.concatenate(
        [sp_ref[0] + g2_ref[0], sp_ref[1] + g2_ref[1]], axis=-1)
    y_ref[...] = jnp.maximum(sval * dinv_ref[...] + b2_ref[...], 0.0)


_tcc = pl.pallas_call(
    _tcc_body,
    grid=(NP // RB,),
    in_specs=[
        pl.BlockSpec((2, RB, DH), lambda i: (0, i, 0)),
        pl.BlockSpec((2, RB, DH), lambda i: (0, i, 0)),
        pl.BlockSpec((RB, 1), lambda i: (i, 0)),
        pl.BlockSpec((1, D), lambda i: (0, 0)),
    ],
    out_specs=pl.BlockSpec((RB, D), lambda i: (i, 0)),
    out_shape=jax.ShapeDtypeStruct((NP, D), jnp.float32),
)


def kernel(x, edge_index, batch, W1, b1, gamma, beta, run_mean, run_var,
           W2, b2):
    src16 = edge_index[0].reshape(NS, NI2, CK)
    dst16 = edge_index[1].reshape(NS, NI2, CK)
    dst32 = edge_index[1].reshape(NW, NIT, CK)

    degp = _deg_kernel(dst32)

    x_pad = jnp.pad(x, ((0, NP - N), (0, 0)))
    dinv, g1 = _tca(degp, x_pad, W1)

    s1 = _scatter_kernel(src16, dst16, g1.reshape(2 * NP, DH))

    a1 = gamma * lax.rsqrt(run_var + EPS)
    c1 = beta - run_mean * a1
    g2 = _tcb(s1, g1, dinv, b1[None, :], a1[None, :], c1[None, :], W2)

    s2 = _scatter_kernel(src16, dst16, g2.reshape(2 * NP, DH))
    y = _tcc(s2, g2, dinv, b2[None, :])
    return y[:N]


# speedup vs baseline: 32.9632x; 1.0389x over previous
import functools

import jax
import jax.numpy as jnp
from jax import lax
from jax.experimental import pallas as pl
from jax.experimental.pallas import tpu as pltpu
from jax.experimental.pallas import tpu_sc as plsc

N = 10000
E = 320000
D = 128
DH = D // 2
EPS = 1e-5

NC = 2
NS = 16
NW = NC * NS
CK = 80
EPW = E // NW
NIT = EPW // CK
EPT = E // NS
NI2 = EPT // CK
NP = 10240
RPT = NP // NS
ZR = 160
RB = 2048
NBUF = 5
DGRP = 25

_mesh = plsc.VectorSubcoreMesh(core_axis_name="c", subcore_axis_name="s")


@functools.partial(
    pl.kernel,
    mesh=_mesh,
    out_type=jax.ShapeDtypeStruct((NC, NP, 16), jnp.float32),
    scratch_types=[
        pltpu.VMEM((NIT, CK), jnp.int32),
        pltpu.VMEM((CK, 16), jnp.float32),
        pltpu.VMEM((RPT, 16), jnp.float32),
        pltpu.VMEM_SHARED((NP, 16), jnp.float32),
        pltpu.SemaphoreType.DMA,
    ],
    compiler_params=pltpu.CompilerParams(use_tc_tiling_on_sc=False),
)
def _deg_kernel(dst_hbm, out_hbm, idx_v, ones_v, buf_v, acc_sh, sem):
    c = lax.axis_index("c")
    s = lax.axis_index("s")
    wid = s * NC + c

    def fill(i, _):
        ones_v[i] = jnp.full((16,), 1.0, jnp.float32)
        return 0

    lax.fori_loop(0, CK, fill, 0)

    def fill2(i, _):
        buf_v[i] = jnp.zeros((16,), jnp.float32)
        return 0

    lax.fori_loop(0, RPT, fill2, 0)

    pltpu.sync_copy(buf_v, acc_sh.at[pl.ds(s * RPT, RPT)])
    plsc.subcore_barrier()

    pltpu.sync_copy(dst_hbm.at[wid], idx_v)

    def dgrp(g, _):
        def fire(j, _):
            pltpu.async_copy(ones_v, acc_sh.at[idx_v.at[j]], sem, add=True)
            return 0

        lax.fori_loop(g * DGRP, (g + 1) * DGRP, fire, 0)

        def drain(j, _):
            pltpu.make_async_copy(ones_v, acc_sh.at[idx_v.at[j]],
                                  sem).wait()
            return 0

        lax.fori_loop(g * DGRP, (g + 1) * DGRP, drain, 0)
        return 0

    lax.fori_loop(0, NIT // DGRP, dgrp, 0)
    plsc.subcore_barrier()

    pltpu.sync_copy(acc_sh.at[pl.ds(s * RPT, RPT)], buf_v)
    pltpu.sync_copy(buf_v, out_hbm.at[c, pl.ds(s * RPT, RPT)])


@functools.partial(
    pl.kernel,
    mesh=_mesh,
    out_type=jax.ShapeDtypeStruct((NC, NP, DH), jnp.float32),
    scratch_types=[
        pltpu.VMEM((NI2, CK), jnp.int32),
        pltpu.VMEM((NI2, CK), jnp.int32),
        pltpu.VMEM((NBUF, CK, DH), jnp.float32),
        pltpu.VMEM((ZR, DH), jnp.float32),
        pltpu.VMEM_SHARED((NP, DH), jnp.float32),
    ] + [pltpu.SemaphoreType.DMA] * NBUF,
    compiler_params=pltpu.CompilerParams(use_tc_tiling_on_sc=False),
)
def _scatter_kernel(src_hbm, dst_hbm, g_hbm, out_hbm, srcv, dstv, rows_v,
                    buf_v, acc_sh, *sems):
    c = lax.axis_index("c")
    s = lax.axis_index("s")

    def fill(i, _):
        for k in range(DH // 16):
            buf_v[i, pl.ds(16 * k, 16)] = jnp.zeros((16,), jnp.float32)
        return 0

    lax.fori_loop(0, ZR, fill, 0)

    for t in range(RPT // ZR):
        pltpu.sync_copy(buf_v, acc_sh.at[pl.ds(s * RPT + t * ZR, ZR)])
    plsc.subcore_barrier()

    pltpu.sync_copy(src_hbm.at[s], srcv)
    pltpu.sync_copy(dst_hbm.at[s], dstv)

    off = c * NP

    def adjust(j, _):
        for k in range(CK // 16):
            sl = pl.ds(16 * k, 16)
            srcv[j, sl] = srcv[j, sl] + off
        return 0

    lax.fori_loop(0, NI2, adjust, 0)

    for b in range(NBUF):
        pltpu.async_copy(g_hbm.at[srcv.at[b]], rows_v.at[b], sems[b])

    def grp(g, _):
        for b in range(NBUF):
            j = g * NBUF + b
            pltpu.make_async_copy(
                g_hbm.at[srcv.at[j]], rows_v.at[b], sems[b]).wait()
            pltpu.sync_copy(rows_v.at[b], acc_sh.at[dstv.at[j]], add=True)
            nj = j + NBUF

            @pl.when(nj < NI2)
            def _():
                pltpu.async_copy(g_hbm.at[srcv.at[nj]], rows_v.at[b],
                                 sems[b])
        return 0

    lax.fori_loop(0, NI2 // NBUF, grp, 0)
    plsc.subcore_barrier()

    for t in range(RPT // ZR):
        sl = pl.ds(s * RPT + t * ZR, ZR)
        pltpu.sync_copy(acc_sh.at[sl], buf_v)
        pltpu.sync_copy(buf_v, out_hbm.at[c, sl])


def _tca_body(degp_ref, x_ref, w_ref, dinv_ref, g_ref):
    deg = degp_ref[0, :, 0:1] + degp_ref[1, :, 0:1] + 1.0
    dinv = lax.rsqrt(deg)
    h = jnp.dot(x_ref[...], w_ref[...], preferred_element_type=jnp.float32)
    g = h * dinv
    dinv_ref[...] = dinv
    g_ref[0] = g[:, :DH]
    g_ref[1] = g[:, DH:]


_tca = pl.pallas_call(
    _tca_body,
    grid=(NP // RB,),
    in_specs=[
        pl.BlockSpec((2, RB, 16), lambda i: (0, i, 0)),
        pl.BlockSpec((RB, D), lambda i: (i, 0)),
        pl.BlockSpec((D, D), lambda i: (0, 0)),
    ],
    out_specs=[
        pl.BlockSpec((RB, 1), lambda i: (i, 0)),
        pl.BlockSpec((2, RB, DH), lambda i: (0, i, 0)),
    ],
    out_shape=[
        jax.ShapeDtypeStruct((NP, 1), jnp.float32),
        jax.ShapeDtypeStruct((2, NP, DH), jnp.float32),
    ],
)


def _tcb_body(sp_ref, g1_ref, dinv_ref, b1_ref, a1_ref, c1_ref, w2_ref,
              g2_ref):
    sval = jnp.concatenate(
        [sp_ref[0] + g1_ref[0], sp_ref[1] + g1_ref[1]], axis=-1)
    dinv = dinv_ref[...]
    o = jnp.maximum(sval * dinv + b1_ref[...], 0.0)
    h = o * a1_ref[...] + c1_ref[...]
    hw = jnp.dot(h, w2_ref[...], preferred_element_type=jnp.float32)
    g2 = hw * dinv
    g2_ref[0] = g2[:, :DH]
    g2_ref[1] = g2[:, DH:]


_tcb = pl.pallas_call(
    _tcb_body,
    grid=(NP // RB,),
    in_specs=[
        pl.BlockSpec((2, RB, DH), lambda i: (0, i, 0)),
        pl.BlockSpec((2, RB, DH), lambda i: (0, i, 0)),
        pl.BlockSpec((RB, 1), lambda i: (i, 0)),
        pl.BlockSpec((1, D), lambda i: (0, 0)),
        pl.BlockSpec((1, D), lambda i: (0, 0)),
        pl.BlockSpec((1, D), lambda i: (0, 0)),
        pl.BlockSpec((D, D), lambda i: (0, 0)),
    ],
    out_specs=pl.BlockSpec((2, RB, DH), lambda i: (0, i, 0)),
    out_shape=jax.ShapeDtypeStruct((2, NP, DH), jnp.float32),
)


def _tcc_body(sp_ref, g2_ref, dinv_ref, b2_ref, y_ref):
    sval = jnp.concatenate(
        [sp_ref[0] + g2_ref[0], sp_ref[1] + g2_ref[1]], axis=-1)
    y_ref[...] = jnp.maximum(sval * dinv_ref[...] + b2_ref[...], 0.0)


_tcc = pl.pallas_call(
    _tcc_body,
    grid=(NP // RB,),
    in_specs=[
        pl.BlockSpec((2, RB, DH), lambda i: (0, i, 0)),
        pl.BlockSpec((2, RB, DH), lambda i: (0, i, 0)),
        pl.BlockSpec((RB, 1), lambda i: (i, 0)),
        pl.BlockSpec((1, D), lambda i: (0, 0)),
    ],
    out_specs=pl.BlockSpec((RB, D), lambda i: (i, 0)),
    out_shape=jax.ShapeDtypeStruct((NP, D), jnp.float32),
)


def kernel(x, edge_index, batch, W1, b1, gamma, beta, run_mean, run_var,
           W2, b2):
    src16 = edge_index[0].reshape(NS, NI2, CK)
    dst16 = edge_index[1].reshape(NS, NI2, CK)
    dst32 = edge_index[1].reshape(NW, NIT, CK)

    degp = _deg_kernel(dst32)

    x_pad = jnp.pad(x, ((0, NP - N), (0, 0)))
    dinv, g1 = _tca(degp, x_pad, W1)

    s1 = _scatter_kernel(src16, dst16, g1.reshape(2 * NP, DH))

    a1 = gamma * lax.rsqrt(run_var + EPS)
    c1 = beta - run_mean * a1
    g2 = _tcb(s1, g1, dinv, b1[None, :], a1[None, :], c1[None, :], W2)

    s2 = _scatter_kernel(src16, dst16, g2.reshape(2 * NP, DH))
    y = _tcc(s2, g2, dinv, b2[None, :])
    return y[:N]


# speedup vs baseline: 38.6902x; 1.1737x over previous
import functools

import jax
import jax.numpy as jnp
from jax import lax
from jax.experimental import pallas as pl
from jax.experimental.pallas import tpu as pltpu
from jax.experimental.pallas import tpu_sc as plsc

N = 10000
E = 320000
D = 128
DH = D // 2
EPS = 1e-5

NC = 2
NS = 16
NW = NC * NS
CK = 80
EPW = E // NW
NIT = EPW // CK
EPT = E // NS
NI2 = EPT // CK
NP = 10240
RPT = NP // NS
ZR = 160
RB = 2048
NBUF = 5
DGRP = 25

_mesh = plsc.VectorSubcoreMesh(core_axis_name="c", subcore_axis_name="s")


@functools.partial(
    pl.kernel,
    mesh=_mesh,
    out_type=jax.ShapeDtypeStruct((NC, NP, 16), jnp.float32),
    scratch_types=[
        pltpu.VMEM((NIT, CK), jnp.int32),
        pltpu.VMEM((CK, 16), jnp.float32),
        pltpu.VMEM((RPT, 16), jnp.float32),
        pltpu.VMEM_SHARED((NP, 16), jnp.float32),
        pltpu.SemaphoreType.DMA,
    ],
    compiler_params=pltpu.CompilerParams(use_tc_tiling_on_sc=False),
)
def _deg_kernel(dst_hbm, out_hbm, idx_v, ones_v, buf_v, acc_sh, sem):
    c = lax.axis_index("c")
    s = lax.axis_index("s")
    wid = s * NC + c

    def fill(i, _):
        ones_v[i] = jnp.full((16,), 1.0, jnp.float32)
        return 0

    lax.fori_loop(0, CK, fill, 0)

    def fill2(i, _):
        buf_v[i] = jnp.zeros((16,), jnp.float32)
        return 0

    lax.fori_loop(0, RPT, fill2, 0)

    pltpu.sync_copy(buf_v, acc_sh.at[pl.ds(s * RPT, RPT)])
    plsc.subcore_barrier()

    pltpu.sync_copy(dst_hbm.at[wid], idx_v)

    def dgrp(g, _):
        def fire(j, _):
            pltpu.async_copy(ones_v, acc_sh.at[idx_v.at[j]], sem, add=True)
            return 0

        lax.fori_loop(g * DGRP, (g + 1) * DGRP, fire, 0)

        def drain(j, _):
            pltpu.make_async_copy(ones_v, acc_sh.at[idx_v.at[j]],
                                  sem).wait()
            return 0

        lax.fori_loop(g * DGRP, (g + 1) * DGRP, drain, 0)
        return 0

    lax.fori_loop(0, NIT // DGRP, dgrp, 0)
    plsc.subcore_barrier()

    pltpu.sync_copy(acc_sh.at[pl.ds(s * RPT, RPT)], buf_v)
    pltpu.sync_copy(buf_v, out_hbm.at[c, pl.ds(s * RPT, RPT)])


@functools.partial(
    pl.kernel,
    mesh=_mesh,
    out_type=jax.ShapeDtypeStruct((NP, D), jnp.float32),
    scratch_types=[
        pltpu.VMEM((NI2, CK), jnp.int32),
        pltpu.VMEM((NI2, CK), jnp.int32),
        pltpu.VMEM((NBUF, CK, DH), jnp.float32),
        pltpu.VMEM((ZR, DH), jnp.float32),
        pltpu.VMEM_SHARED((NP, DH), jnp.float32),
    ] + [pltpu.SemaphoreType.DMA] * NBUF,
    compiler_params=pltpu.CompilerParams(use_tc_tiling_on_sc=False),
)
def _scatter_kernel(src_hbm, dst_hbm, g_hbm, out_hbm, srcv, dstv, rows_v,
                    buf_v, acc_sh, *sems):
    c = lax.axis_index("c")
    s = lax.axis_index("s")

    def fill(i, _):
        for k in range(DH // 16):
            buf_v[i, pl.ds(16 * k, 16)] = jnp.zeros((16,), jnp.float32)
        return 0

    lax.fori_loop(0, ZR, fill, 0)

    for t in range(RPT // ZR):
        pltpu.sync_copy(buf_v, acc_sh.at[pl.ds(s * RPT + t * ZR, ZR)])
    plsc.subcore_barrier()

    pltpu.sync_copy(src_hbm.at[s], srcv)
    pltpu.sync_copy(dst_hbm.at[s], dstv)

    def adjust(j, _):
        for k in range(CK // 16):
            sl = pl.ds(16 * k, 16)
            srcv[j, sl] = srcv[j, sl] * 2 + c
        return 0

    lax.fori_loop(0, NI2, adjust, 0)

    for b in range(NBUF):
        pltpu.async_copy(g_hbm.at[srcv.at[b]], rows_v.at[b], sems[b])

    def grp(g, _):
        for b in range(NBUF):
            j = g * NBUF + b
            pltpu.make_async_copy(
                g_hbm.at[srcv.at[j]], rows_v.at[b], sems[b]).wait()
            pltpu.sync_copy(rows_v.at[b], acc_sh.at[dstv.at[j]], add=True)
            nj = j + NBUF

            @pl.when(nj < NI2)
            def _():
                pltpu.async_copy(g_hbm.at[srcv.at[nj]], rows_v.at[b],
                                 sems[b])
        return 0

    lax.fori_loop(0, NI2 // NBUF, grp, 0)
    plsc.subcore_barrier()

    for t in range(RPT // ZR):
        sl = pl.ds(s * RPT + t * ZR, ZR)
        pltpu.sync_copy(acc_sh.at[sl], buf_v)
        pltpu.sync_copy(buf_v, out_hbm.at[sl, pl.ds(c * DH, DH)])


def _tca_body(degp_ref, x_ref, w_ref, dinv_ref, g_ref):
    deg = degp_ref[0, :, 0:1] + degp_ref[1, :, 0:1] + 1.0
    dinv = lax.rsqrt(deg)
    h = jnp.dot(x_ref[...], w_ref[...], preferred_element_type=jnp.float32)
    dinv_ref[...] = dinv
    g_ref[...] = h * dinv


_tca = pl.pallas_call(
    _tca_body,
    grid=(NP // RB,),
    in_specs=[
        pl.BlockSpec((2, RB, 16), lambda i: (0, i, 0)),
        pl.BlockSpec((RB, D), lambda i: (i, 0)),
        pl.BlockSpec((D, D), lambda i: (0, 0)),
    ],
    out_specs=[
        pl.BlockSpec((RB, 1), lambda i: (i, 0)),
        pl.BlockSpec((RB, D), lambda i: (i, 0)),
    ],
    out_shape=[
        jax.ShapeDtypeStruct((NP, 1), jnp.float32),
        jax.ShapeDtypeStruct((NP, D), jnp.float32),
    ],
)


def _tcb_body(sp_ref, g1_ref, dinv_ref, b1_ref, a1_ref, c1_ref, w2_ref,
              g2_ref):
    sval = sp_ref[...] + g1_ref[...]
    dinv = dinv_ref[...]
    o = jnp.maximum(sval * dinv + b1_ref[...], 0.0)
    h = o * a1_ref[...] + c1_ref[...]
    g2_ref[...] = jnp.dot(h, w2_ref[...],
                          preferred_element_type=jnp.float32) * dinv


_tcb = pl.pallas_call(
    _tcb_body,
    grid=(NP // RB,),
    in_specs=[
        pl.BlockSpec((RB, D), lambda i: (i, 0)),
        pl.BlockSpec((RB, D), lambda i: (i, 0)),
        pl.BlockSpec((RB, 1), lambda i: (i, 0)),
        pl.BlockSpec((1, D), lambda i: (0, 0)),
        pl.BlockSpec((1, D), lambda i: (0, 0)),
        pl.BlockSpec((1, D), lambda i: (0, 0)),
        pl.BlockSpec((D, D), lambda i: (0, 0)),
    ],
    out_specs=pl.BlockSpec((RB, D), lambda i: (i, 0)),
    out_shape=jax.ShapeDtypeStruct((NP, D), jnp.float32),
)


def _tcc_body(sp_ref, g2_ref, dinv_ref, b2_ref, y_ref):
    sval = sp_ref[...] + g2_ref[...]
    y_ref[...] = jnp.maximum(sval * dinv_ref[...] + b2_ref[...], 0.0)


_tcc = pl.pallas_call(
    _tcc_body,
    grid=(NP // RB,),
    in_specs=[
        pl.BlockSpec((RB, D), lambda i: (i, 0)),
        pl.BlockSpec((RB, D), lambda i: (i, 0)),
        pl.BlockSpec((RB, 1), lambda i: (i, 0)),
        pl.BlockSpec((1, D), lambda i: (0, 0)),
    ],
    out_specs=pl.BlockSpec((RB, D), lambda i: (i, 0)),
    out_shape=jax.ShapeDtypeStruct((NP, D), jnp.float32),
)


def kernel(x, edge_index, batch, W1, b1, gamma, beta, run_mean, run_var,
           W2, b2):
    src16 = edge_index[0].reshape(NS, NI2, CK)
    dst16 = edge_index[1].reshape(NS, NI2, CK)
    dst32 = edge_index[1].reshape(NW, NIT, CK)

    degp = _deg_kernel(dst32)

    x_pad = jnp.pad(x, ((0, NP - N), (0, 0)))
    dinv, g1 = _tca(degp, x_pad, W1)

    s1 = _scatter_kernel(src16, dst16, g1.reshape(2 * NP, DH))

    a1 = gamma * lax.rsqrt(run_var + EPS)
    c1 = beta - run_mean * a1
    g2 = _tcb(s1, g1, dinv, b1[None, :], a1[None, :], c1[None, :], W2)

    s2 = _scatter_kernel(src16, dst16, g2.reshape(2 * NP, DH))
    y = _tcc(s2, g2, dinv, b2[None, :])
    return y[:N]


# speedup vs baseline: 38.7872x; 1.0025x over previous
import functools

import jax
import jax.numpy as jnp
from jax import lax
from jax.experimental import pallas as pl
from jax.experimental.pallas import tpu as pltpu
from jax.experimental.pallas import tpu_sc as plsc

N = 10000
E = 320000
D = 128
DH = D // 2
EPS = 1e-5

NC = 2
NS = 16
NW = NC * NS
CK = 80
EPW = E // NW
NIT = EPW // CK
EPT = E // NS
NI2 = EPT // CK
NP = 10240
RPT = NP // NS
ZR = 160
RB = 2048
NBUF = 5
DGRP = 25

_mesh = plsc.VectorSubcoreMesh(core_axis_name="c", subcore_axis_name="s")


@functools.partial(
    pl.kernel,
    mesh=_mesh,
    out_type=jax.ShapeDtypeStruct((NC, NP, D), jnp.float32),
    scratch_types=[
        pltpu.VMEM((NIT, CK), jnp.int32),
        pltpu.VMEM((CK, 16), jnp.float32),
        pltpu.VMEM((RPT, 16), jnp.float32),
        pltpu.VMEM_SHARED((NP, 16), jnp.float32),
        pltpu.SemaphoreType.DMA,
    ],
    compiler_params=pltpu.CompilerParams(use_tc_tiling_on_sc=False),
)
def _deg_kernel(dst_hbm, out_hbm, idx_v, ones_v, buf_v, acc_sh, sem):
    c = lax.axis_index("c")
    s = lax.axis_index("s")
    wid = s * NC + c

    def fill(i, _):
        ones_v[i] = jnp.full((16,), 1.0, jnp.float32)
        return 0

    lax.fori_loop(0, CK, fill, 0)

    def fill2(i, _):
        buf_v[i] = jnp.zeros((16,), jnp.float32)
        return 0

    lax.fori_loop(0, RPT, fill2, 0)

    pltpu.sync_copy(buf_v, acc_sh.at[pl.ds(s * RPT, RPT)])
    plsc.subcore_barrier()

    pltpu.sync_copy(dst_hbm.at[wid], idx_v)

    def dgrp(g, _):
        def fire(j, _):
            pltpu.async_copy(ones_v, acc_sh.at[idx_v.at[j]], sem, add=True)
            return 0

        lax.fori_loop(g * DGRP, (g + 1) * DGRP, fire, 0)

        def drain(j, _):
            pltpu.make_async_copy(ones_v, acc_sh.at[idx_v.at[j]],
                                  sem).wait()
            return 0

        lax.fori_loop(g * DGRP, (g + 1) * DGRP, drain, 0)
        return 0

    lax.fori_loop(0, NIT // DGRP, dgrp, 0)
    plsc.subcore_barrier()

    pltpu.sync_copy(acc_sh.at[pl.ds(s * RPT, RPT)], buf_v)
    pltpu.sync_copy(buf_v, out_hbm.at[c, pl.ds(s * RPT, RPT), pl.ds(0, 16)])


@functools.partial(
    pl.kernel,
    mesh=_mesh,
    out_type=jax.ShapeDtypeStruct((NP, D), jnp.float32),
    scratch_types=[
        pltpu.VMEM((NI2, CK), jnp.int32),
        pltpu.VMEM((NI2, CK), jnp.int32),
        pltpu.VMEM((NBUF, CK, DH), jnp.float32),
        pltpu.VMEM((ZR, DH), jnp.float32),
        pltpu.VMEM_SHARED((NP, DH), jnp.float32),
    ] + [pltpu.SemaphoreType.DMA] * (2 * NBUF),
    compiler_params=pltpu.CompilerParams(use_tc_tiling_on_sc=False),
)
def _scatter_kernel(src_hbm, dst_hbm, g_hbm, out_hbm, srcv, dstv, rows_v,
                    buf_v, acc_sh, *sems):
    c = lax.axis_index("c")
    s = lax.axis_index("s")

    def fill(i, _):
        for k in range(DH // 16):
            buf_v[i, pl.ds(16 * k, 16)] = jnp.zeros((16,), jnp.float32)
        return 0

    lax.fori_loop(0, ZR, fill, 0)

    for t in range(RPT // ZR):
        pltpu.sync_copy(buf_v, acc_sh.at[pl.ds(s * RPT + t * ZR, ZR)])
    plsc.subcore_barrier()

    pltpu.sync_copy(src_hbm.at[s], srcv)
    pltpu.sync_copy(dst_hbm.at[s], dstv)

    def adjust(j, _):
        for k in range(CK // 16):
            sl = pl.ds(16 * k, 16)
            srcv[j, sl] = srcv[j, sl] * 2 + c
        return 0

    lax.fori_loop(0, NI2, adjust, 0)

    gsems = sems[:NBUF]
    ssems = sems[NBUF:]

    for b in range(NBUF):
        pltpu.async_copy(g_hbm.at[srcv.at[b]], rows_v.at[b], gsems[b])

    def grp(g, _):
        for b in range(NBUF):
            j = g * NBUF + b
            bp = (b - 1) % NBUF
            pltpu.make_async_copy(
                g_hbm.at[srcv.at[j]], rows_v.at[b], gsems[b]).wait()
            pltpu.async_copy(rows_v.at[b], acc_sh.at[dstv.at[j]], ssems[b],
                             add=True)
            pj = j + NBUF - 1

            def reissue():
                pltpu.make_async_copy(
                    rows_v.at[bp], acc_sh.at[dstv.at[j - 1]],
                    ssems[bp]).wait()
                pltpu.async_copy(g_hbm.at[srcv.at[pj]], rows_v.at[bp],
                                 gsems[bp])

            if b == 0:
                pl.when(jnp.logical_and(j >= 1, pj < NI2))(reissue)
            else:
                pl.when(pj < NI2)(reissue)
        return 0

    lax.fori_loop(0, NI2 // NBUF, grp, 0)

    last = NI2 - NBUF
    for b in range(NBUF):
        pltpu.make_async_copy(
            rows_v.at[b], acc_sh.at[dstv.at[last + b]], ssems[b]).wait()
    plsc.subcore_barrier()

    for t in range(RPT // ZR):
        sl = pl.ds(s * RPT + t * ZR, ZR)
        pltpu.sync_copy(acc_sh.at[sl], buf_v)
        pltpu.sync_copy(buf_v, out_hbm.at[sl, pl.ds(c * DH, DH)])


def _tca_body(degp_ref, x_ref, w_ref, dinv_ref, g_ref):
    deg = degp_ref[0, :, 0:1] + degp_ref[1, :, 0:1] + 1.0
    dinv = lax.rsqrt(deg)
    h = jnp.dot(x_ref[...], w_ref[...], preferred_element_type=jnp.float32)
    dinv_ref[...] = dinv
    g_ref[...] = h * dinv


_tca = pl.pallas_call(
    _tca_body,
    grid=(NP // RB,),
    in_specs=[
        pl.BlockSpec((2, RB, D), lambda i: (0, i, 0)),
        pl.BlockSpec((RB, D), lambda i: (i, 0)),
        pl.BlockSpec((D, D), lambda i: (0, 0)),
    ],
    out_specs=[
        pl.BlockSpec((RB, 1), lambda i: (i, 0)),
        pl.BlockSpec((RB, D), lambda i: (i, 0)),
    ],
    out_shape=[
        jax.ShapeDtypeStruct((NP, 1), jnp.float32),
        jax.ShapeDtypeStruct((NP, D), jnp.float32),
    ],
)


def _tcb_body(sp_ref, g1_ref, dinv_ref, b1_ref, a1_ref, c1_ref, w2_ref,
              g2_ref):
    sval = sp_ref[...] + g1_ref[...]
    dinv = dinv_ref[...]
    o = jnp.maximum(sval * dinv + b1_ref[...], 0.0)
    h = o * a1_ref[...] + c1_ref[...]
    g2_ref[...] = jnp.dot(h, w2_ref[...],
                          preferred_element_type=jnp.float32) * dinv


_tcb = pl.pallas_call(
    _tcb_body,
    grid=(NP // RB,),
    in_specs=[
        pl.BlockSpec((RB, D), lambda i: (i, 0)),
        pl.BlockSpec((RB, D), lambda i: (i, 0)),
        pl.BlockSpec((RB, 1), lambda i: (i, 0)),
        pl.BlockSpec((1, D), lambda i: (0, 0)),
        pl.BlockSpec((1, D), lambda i: (0, 0)),
        pl.BlockSpec((1, D), lambda i: (0, 0)),
        pl.BlockSpec((D, D), lambda i: (0, 0)),
    ],
    out_specs=pl.BlockSpec((RB, D), lambda i: (i, 0)),
    out_shape=jax.ShapeDtypeStruct((NP, D), jnp.float32),
)


def _tcc_body(sp_ref, g2_ref, dinv_ref, b2_ref, y_ref):
    sval = sp_ref[...] + g2_ref[...]
    y_ref[...] = jnp.maximum(sval * dinv_ref[...] + b2_ref[...], 0.0)


_tcc = pl.pallas_call(
    _tcc_body,
    grid=(NP // RB,),
    in_specs=[
        pl.BlockSpec((RB, D), lambda i: (i, 0)),
        pl.BlockSpec((RB, D), lambda i: (i, 0)),
        pl.BlockSpec((RB, 1), lambda i: (i, 0)),
        pl.BlockSpec((1, D), lambda i: (0, 0)),
    ],
    out_specs=pl.BlockSpec((RB, D), lambda i: (i, 0)),
    out_shape=jax.ShapeDtypeStruct((NP, D), jnp.float32),
)


def kernel(x, edge_index, batch, W1, b1, gamma, beta, run_mean, run_var,
           W2, b2):
    src16 = edge_index[0].reshape(NS, NI2, CK)
    dst16 = edge_index[1].reshape(NS, NI2, CK)
    dst32 = edge_index[1].reshape(NW, NIT, CK)

    degp = _deg_kernel(dst32)

    x_pad = jnp.pad(x, ((0, NP - N), (0, 0)))
    dinv, g1 = _tca(degp, x_pad, W1)

    s1 = _scatter_kernel(src16, dst16, g1.reshape(2 * NP, DH))

    a1 = gamma * lax.rsqrt(run_var + EPS)
    c1 = beta - run_mean * a1
    g2 = _tcb(s1, g1, dinv, b1[None, :], a1[None, :], c1[None, :], W2)

    s2 = _scatter_kernel(src16, dst16, g2.reshape(2 * NP, DH))
    y = _tcc(s2, g2, dinv, b2[None, :])
    return y[:N]


# speedup vs baseline: 39.4877x; 1.0181x over previous
import functools

import jax
import jax.numpy as jnp
from jax import lax
from jax.experimental import pallas as pl
from jax.experimental.pallas import tpu as pltpu
from jax.experimental.pallas import tpu_sc as plsc

N = 10000
E = 320000
D = 128
DH = D // 2
EPS = 1e-5

NC = 2
NS = 16
NW = NC * NS
CK = 80
EPW = E // NW
NIT = EPW // CK
EPT = E // NS
NI2 = EPT // CK
NP = 10240
RPT = NP // NS
ZR = 160
RB = 2048
NBUF = 5
DGRP = 25

_mesh = plsc.VectorSubcoreMesh(core_axis_name="c", subcore_axis_name="s")


@functools.partial(
    pl.kernel,
    mesh=_mesh,
    out_type=jax.ShapeDtypeStruct((NC, NP, D), jnp.float32),
    scratch_types=[
        pltpu.VMEM((NIT, CK), jnp.int32),
        pltpu.VMEM((CK, 16), jnp.float32),
        pltpu.VMEM((RPT, 16), jnp.float32),
        pltpu.VMEM_SHARED((NP, 16), jnp.float32),
        pltpu.SemaphoreType.DMA,
    ],
    compiler_params=pltpu.CompilerParams(use_tc_tiling_on_sc=False),
)
def _deg_kernel(dst_hbm, out_hbm, idx_v, ones_v, buf_v, acc_sh, sem):
    c = lax.axis_index("c")
    s = lax.axis_index("s")
    wid = s * NC + c

    def fill(i, _):
        ones_v[i] = jnp.full((16,), 1.0, jnp.float32)
        return 0

    lax.fori_loop(0, CK, fill, 0)

    def fill2(i, _):
        buf_v[i] = jnp.zeros((16,), jnp.float32)
        return 0

    lax.fori_loop(0, RPT, fill2, 0)

    pltpu.sync_copy(buf_v, acc_sh.at[pl.ds(s * RPT, RPT)])
    plsc.subcore_barrier()

    pltpu.sync_copy(dst_hbm.at[wid], idx_v)

    def dgrp(g, _):
        def fire(j, _):
            pltpu.async_copy(ones_v, acc_sh.at[idx_v.at[j]], sem, add=True)
            return 0

        lax.fori_loop(g * DGRP, (g + 1) * DGRP, fire, 0)

        def drain(j, _):
            pltpu.make_async_copy(ones_v, acc_sh.at[idx_v.at[j]],
                                  sem).wait()
            return 0

        lax.fori_loop(g * DGRP, (g + 1) * DGRP, drain, 0)
        return 0

    lax.fori_loop(0, NIT // DGRP, dgrp, 0)
    plsc.subcore_barrier()

    pltpu.sync_copy(acc_sh.at[pl.ds(s * RPT, RPT)], buf_v)
    pltpu.sync_copy(buf_v, out_hbm.at[c, pl.ds(s * RPT, RPT), pl.ds(0, 16)])


@functools.partial(
    pl.kernel,
    mesh=_mesh,
    out_type=jax.ShapeDtypeStruct((NP, D), jnp.float32),
    scratch_types=[
        pltpu.VMEM((NI2, CK), jnp.int32),
        pltpu.VMEM((NI2, CK), jnp.int32),
        pltpu.VMEM((NBUF, CK, DH), jnp.float32),
        pltpu.VMEM((ZR, DH), jnp.float32),
        pltpu.VMEM_SHARED((NP, DH), jnp.float32),
    ] + [pltpu.SemaphoreType.DMA] * (2 * NBUF),
    compiler_params=pltpu.CompilerParams(use_tc_tiling_on_sc=False),
)
def _scatter_kernel(src_hbm, dst_hbm, g_hbm, out_hbm, srcv, dstv, rows_v,
                    buf_v, acc_sh, *sems):
    c = lax.axis_index("c")
    s = lax.axis_index("s")

    def fill(i, _):
        for k in range(DH // 16):
            buf_v[i, pl.ds(16 * k, 16)] = jnp.zeros((16,), jnp.float32)
        return 0

    lax.fori_loop(0, ZR, fill, 0)

    for t in range(RPT // ZR):
        pltpu.sync_copy(buf_v, acc_sh.at[pl.ds(s * RPT + t * ZR, ZR)])
    plsc.subcore_barrier()

    pltpu.sync_copy(src_hbm.at[s], srcv)
    pltpu.sync_copy(dst_hbm.at[s], dstv)

    def adjust(j, _):
        for k in range(CK // 16):
            sl = pl.ds(16 * k, 16)
            srcv[j, sl] = srcv[j, sl] * 2 + c
        return 0

    lax.fori_loop(0, NI2, adjust, 0)

    gsems = sems[:NBUF]
    ssems = sems[NBUF:]

    for b in range(NBUF):
        pltpu.async_copy(g_hbm.at[srcv.at[b]], rows_v.at[b], gsems[b])

    def grp(g, _):
        for b in range(NBUF):
            j = g * NBUF + b
            bp = (b - 1) % NBUF
            pltpu.make_async_copy(
                g_hbm.at[srcv.at[j]], rows_v.at[b], gsems[b]).wait()
            pltpu.async_copy(rows_v.at[b], acc_sh.at[dstv.at[j]], ssems[b],
                             add=True)
            pj = j + NBUF - 1

            def reissue():
                pltpu.make_async_copy(
                    rows_v.at[bp], acc_sh.at[dstv.at[j - 1]],
                    ssems[bp]).wait()
                pltpu.async_copy(g_hbm.at[srcv.at[pj]], rows_v.at[bp],
                                 gsems[bp])

            if b == 0:
                pl.when(jnp.logical_and(j >= 1, pj < NI2))(reissue)
            else:
                pl.when(pj < NI2)(reissue)
        return 0

    lax.fori_loop(0, NI2 // NBUF, grp, 0)

    last = NI2 - NBUF
    for b in range(NBUF):
        pltpu.make_async_copy(
            rows_v.at[b], acc_sh.at[dstv.at[last + b]], ssems[b]).wait()
    plsc.subcore_barrier()

    for t in range(RPT // ZR):
        sl = pl.ds(s * RPT + t * ZR, ZR)
        pltpu.sync_copy(acc_sh.at[sl], buf_v)
        pltpu.sync_copy(buf_v, out_hbm.at[sl, pl.ds(c * DH, DH)])


def _tca_body(degp_ref, x_ref, w_ref, dinv_ref, g_ref):
    deg = degp_ref[0, :, 0:1] + degp_ref[1, :, 0:1] + 1.0
    dinv = lax.rsqrt(deg)
    h = jnp.dot(x_ref[...], w_ref[...], preferred_element_type=jnp.float32)
    dinv_ref[...] = dinv
    g_ref[...] = h * dinv


_tca = pl.pallas_call(
    _tca_body,
    grid=(NP // RB,),
    in_specs=[
        pl.BlockSpec((2, RB, D), lambda i: (0, i, 0)),
        pl.BlockSpec((RB, D), lambda i: (i, 0)),
        pl.BlockSpec((D, D), lambda i: (0, 0)),
    ],
    out_specs=[
        pl.BlockSpec((RB, 1), lambda i: (i, 0)),
        pl.BlockSpec((RB, D), lambda i: (i, 0)),
    ],
    out_shape=[
        jax.ShapeDtypeStruct((NP, 1), jnp.float32),
        jax.ShapeDtypeStruct((NP, D), jnp.float32),
    ],
)


def _tcb_body(sp_ref, g1_ref, dinv_ref, b1_ref, a1_ref, c1_ref, w2_ref,
              g2_ref):
    sval = sp_ref[...] + g1_ref[...]
    dinv = dinv_ref[...]
    o = jnp.maximum(sval * dinv + b1_ref[...], 0.0)
    h = o * a1_ref[...] + c1_ref[...]
    g2_ref[...] = jnp.dot(h, w2_ref[...],
                          preferred_element_type=jnp.float32) * dinv


_tcb = pl.pallas_call(
    _tcb_body,
    grid=(NP // RB,),
    in_specs=[
        pl.BlockSpec((RB, D), lambda i: (i, 0)),
        pl.BlockSpec((RB, D), lambda i: (i, 0)),
        pl.BlockSpec((RB, 1), lambda i: (i, 0)),
        pl.BlockSpec((1, D), lambda i: (0, 0)),
        pl.BlockSpec((1, D), lambda i: (0, 0)),
        pl.BlockSpec((1, D), lambda i: (0, 0)),
        pl.BlockSpec((D, D), lambda i: (0, 0)),
    ],
    out_specs=pl.BlockSpec((RB, D), lambda i: (i, 0)),
    out_shape=jax.ShapeDtypeStruct((NP, D), jnp.float32),
)


def _tcc_body(sp_ref, g2_ref, dinv_ref, b2_ref, y_ref):
    sval = sp_ref[...] + g2_ref[...]
    y_ref[...] = jnp.maximum(sval * dinv_ref[...] + b2_ref[...], 0.0)


_tcc = pl.pallas_call(
    _tcc_body,
    grid=(NP // RB,),
    in_specs=[
        pl.BlockSpec((RB, D), lambda i: (i, 0)),
        pl.BlockSpec((RB, D), lambda i: (i, 0)),
        pl.BlockSpec((RB, 1), lambda i: (i, 0)),
        pl.BlockSpec((1, D), lambda i: (0, 0)),
    ],
    out_specs=pl.BlockSpec((RB, D), lambda i: (i, 0)),
    out_shape=jax.ShapeDtypeStruct((N, D), jnp.float32),
)


def kernel(x, edge_index, batch, W1, b1, gamma, beta, run_mean, run_var,
           W2, b2):
    src16 = edge_index[0].reshape(NS, NI2, CK)
    dst16 = edge_index[1].reshape(NS, NI2, CK)
    dst32 = edge_index[1].reshape(NW, NIT, CK)

    degp = _deg_kernel(dst32)

    dinv, g1 = _tca(degp, x, W1)

    s1 = _scatter_kernel(src16, dst16, g1.reshape(2 * NP, DH))

    a1 = gamma * lax.rsqrt(run_var + EPS)
    c1 = beta - run_mean * a1
    g2 = _tcb(s1, g1, dinv, b1[None, :], a1[None, :], c1[None, :], W2)

    s2 = _scatter_kernel(src16, dst16, g2.reshape(2 * NP, DH))
    return _tcc(s2, g2, dinv, b2[None, :])


# speedup vs baseline: 41.3001x; 1.0459x over previous
import functools

import jax
import jax.numpy as jnp
from jax import lax
from jax.experimental import pallas as pl
from jax.experimental.pallas import tpu as pltpu
from jax.experimental.pallas import tpu_sc as plsc

N = 10000
E = 320000
D = 128
DH = D // 2
EPS = 1e-5

NC = 2
NS = 16
NW = NC * NS
CK = 80
ECH = E // 128
DCH = ECH // NW
DXT = ECH - DCH * NW
EPT = E // NS
NI2 = EPT // CK
NP = 10240
RPT = NP // NS
ZR = 160
RB = 2048
NBUF = 5
DGRP = 25

_mesh = plsc.VectorSubcoreMesh(core_axis_name="c", subcore_axis_name="s")


@functools.partial(
    pl.kernel,
    mesh=_mesh,
    out_type=jax.ShapeDtypeStruct((NC, NP, D), jnp.float32),
    scratch_types=[
        pltpu.VMEM((DCH + 1, 128), jnp.int32),
        pltpu.VMEM((128, 16), jnp.float32),
        pltpu.VMEM((RPT, 16), jnp.float32),
        pltpu.VMEM_SHARED((NP, 16), jnp.float32),
        pltpu.SemaphoreType.DMA,
    ],
    compiler_params=pltpu.CompilerParams(use_tc_tiling_on_sc=False),
)
def _deg_kernel(ei_hbm, out_hbm, idx_v, ones_v, buf_v, acc_sh, sem):
    c = lax.axis_index("c")
    s = lax.axis_index("s")
    wid = s * NC + c

    def fill(i, _):
        ones_v[i] = jnp.full((16,), 1.0, jnp.float32)
        return 0

    lax.fori_loop(0, 128, fill, 0)

    def fill2(i, _):
        buf_v[i] = jnp.zeros((16,), jnp.float32)
        return 0

    lax.fori_loop(0, RPT, fill2, 0)

    pltpu.sync_copy(buf_v, acc_sh.at[pl.ds(s * RPT, RPT)])
    plsc.subcore_barrier()

    pltpu.sync_copy(ei_hbm.at[pl.ds(wid * DCH, DCH), 1],
                    idx_v.at[pl.ds(0, DCH)])
    extra = wid < DXT

    @pl.when(extra)
    def _():
        pltpu.sync_copy(ei_hbm.at[NW * DCH + wid, 1], idx_v.at[DCH])

    nch = jnp.where(extra, DCH + 1, DCH)

    def fire(j, _):
        pltpu.async_copy(ones_v, acc_sh.at[idx_v.at[j]], sem, add=True)
        return 0

    def drain(j, _):
        pltpu.make_async_copy(ones_v, acc_sh.at[idx_v.at[j]], sem).wait()
        return 0

    def dgrp(g, _):
        lax.fori_loop(g * DGRP, (g + 1) * DGRP, fire, 0)
        lax.fori_loop(g * DGRP, (g + 1) * DGRP, drain, 0)
        return 0

    ngrp = DCH // DGRP
    lax.fori_loop(0, ngrp, dgrp, 0)
    lax.fori_loop(ngrp * DGRP, nch, fire, 0)
    lax.fori_loop(ngrp * DGRP, nch, drain, 0)
    plsc.subcore_barrier()

    pltpu.sync_copy(acc_sh.at[pl.ds(s * RPT, RPT)], buf_v)
    pltpu.sync_copy(buf_v, out_hbm.at[c, pl.ds(s * RPT, RPT), pl.ds(0, 16)])


@functools.partial(
    pl.kernel,
    mesh=_mesh,
    out_type=jax.ShapeDtypeStruct((NP, D), jnp.float32),
    scratch_types=[
        pltpu.VMEM((NI2, CK), jnp.int32),
        pltpu.VMEM((NI2, CK), jnp.int32),
        pltpu.VMEM((NBUF, CK, DH), jnp.float32),
        pltpu.VMEM((ZR, DH), jnp.float32),
        pltpu.VMEM_SHARED((NP, DH), jnp.float32),
    ] + [pltpu.SemaphoreType.DMA] * (2 * NBUF),
    compiler_params=pltpu.CompilerParams(use_tc_tiling_on_sc=False),
)
def _scatter_kernel(src_hbm, dst_hbm, g_hbm, out_hbm, srcv, dstv, rows_v,
                    buf_v, acc_sh, *sems):
    c = lax.axis_index("c")
    s = lax.axis_index("s")

    def fill(i, _):
        for k in range(DH // 16):
            buf_v[i, pl.ds(16 * k, 16)] = jnp.zeros((16,), jnp.float32)
        return 0

    lax.fori_loop(0, ZR, fill, 0)

    for t in range(RPT // ZR):
        pltpu.sync_copy(buf_v, acc_sh.at[pl.ds(s * RPT + t * ZR, ZR)])
    plsc.subcore_barrier()

    pltpu.sync_copy(src_hbm.at[s], srcv)
    pltpu.sync_copy(dst_hbm.at[s], dstv)

    def adjust(j, _):
        for k in range(CK // 16):
            sl = pl.ds(16 * k, 16)
            srcv[j, sl] = srcv[j, sl] * 2 + c
        return 0

    lax.fori_loop(0, NI2, adjust, 0)

    gsems = sems[:NBUF]
    ssems = sems[NBUF:]

    for b in range(NBUF):
        pltpu.async_copy(g_hbm.at[srcv.at[b]], rows_v.at[b], gsems[b])

    def grp(g, _):
        for b in range(NBUF):
            j = g * NBUF + b
            bp = (b - 1) % NBUF
            pltpu.make_async_copy(
                g_hbm.at[srcv.at[j]], rows_v.at[b], gsems[b]).wait()
            pltpu.async_copy(rows_v.at[b], acc_sh.at[dstv.at[j]], ssems[b],
                             add=True)
            pj = j + NBUF - 1

            def reissue():
                pltpu.make_async_copy(
                    rows_v.at[bp], acc_sh.at[dstv.at[j - 1]],
                    ssems[bp]).wait()
                pltpu.async_copy(g_hbm.at[srcv.at[pj]], rows_v.at[bp],
                                 gsems[bp])

            if b == 0:
                pl.when(jnp.logical_and(j >= 1, pj < NI2))(reissue)
            else:
                pl.when(pj < NI2)(reissue)
        return 0

    lax.fori_loop(0, NI2 // NBUF, grp, 0)

    last = NI2 - NBUF
    for b in range(NBUF):
        pltpu.make_async_copy(
            rows_v.at[b], acc_sh.at[dstv.at[last + b]], ssems[b]).wait()
    plsc.subcore_barrier()

    for t in range(RPT // ZR):
        sl = pl.ds(s * RPT + t * ZR, ZR)
        pltpu.sync_copy(acc_sh.at[sl], buf_v)
        pltpu.sync_copy(buf_v, out_hbm.at[sl, pl.ds(c * DH, DH)])


def _tca_body(degp_ref, x_ref, w_ref, dinv_ref, g_ref):
    deg = degp_ref[0, :, 0:1] + degp_ref[1, :, 0:1] + 1.0
    dinv = lax.rsqrt(deg)
    h = jnp.dot(x_ref[...], w_ref[...], preferred_element_type=jnp.float32)
    dinv_ref[...] = dinv
    g_ref[...] = h * dinv


_tca = pl.pallas_call(
    _tca_body,
    grid=(NP // RB,),
    in_specs=[
        pl.BlockSpec((2, RB, D), lambda i: (0, i, 0)),
        pl.BlockSpec((RB, D), lambda i: (i, 0)),
        pl.BlockSpec((D, D), lambda i: (0, 0)),
    ],
    out_specs=[
        pl.BlockSpec((RB, 1), lambda i: (i, 0)),
        pl.BlockSpec((RB, D), lambda i: (i, 0)),
    ],
    out_shape=[
        jax.ShapeDtypeStruct((NP, 1), jnp.float32),
        jax.ShapeDtypeStruct((NP, D), jnp.float32),
    ],
)


def _tcb_body(sp_ref, g1_ref, dinv_ref, b1_ref, a1_ref, c1_ref, w2_ref,
              g2_ref):
    sval = sp_ref[...] + g1_ref[...]
    dinv = dinv_ref[...]
    o = jnp.maximum(sval * dinv + b1_ref[...], 0.0)
    h = o * a1_ref[...] + c1_ref[...]
    g2_ref[...] = jnp.dot(h, w2_ref[...],
                          preferred_element_type=jnp.float32) * dinv


_tcb = pl.pallas_call(
    _tcb_body,
    grid=(NP // RB,),
    in_specs=[
        pl.BlockSpec((RB, D), lambda i: (i, 0)),
        pl.BlockSpec((RB, D), lambda i: (i, 0)),
        pl.BlockSpec((RB, 1), lambda i: (i, 0)),
        pl.BlockSpec((1, D), lambda i: (0, 0)),
        pl.BlockSpec((1, D), lambda i: (0, 0)),
        pl.BlockSpec((1, D), lambda i: (0, 0)),
        pl.BlockSpec((D, D), lambda i: (0, 0)),
    ],
    out_specs=pl.BlockSpec((RB, D), lambda i: (i, 0)),
    out_shape=jax.ShapeDtypeStruct((NP, D), jnp.float32),
)


def _tcc_body(sp_ref, g2_ref, dinv_ref, b2_ref, y_ref):
    sval = sp_ref[...] + g2_ref[...]
    y_ref[...] = jnp.maximum(sval * dinv_ref[...] + b2_ref[...], 0.0)


_tcc = pl.pallas_call(
    _tcc_body,
    grid=(NP // RB,),
    in_specs=[
        pl.BlockSpec((RB, D), lambda i: (i, 0)),
        pl.BlockSpec((RB, D), lambda i: (i, 0)),
        pl.BlockSpec((RB, 1), lambda i: (i, 0)),
        pl.BlockSpec((1, D), lambda i: (0, 0)),
    ],
    out_specs=pl.BlockSpec((RB, D), lambda i: (i, 0)),
    out_shape=jax.ShapeDtypeStruct((N, D), jnp.float32),
)


def kernel(x, edge_index, batch, W1, b1, gamma, beta, run_mean, run_var,
           W2, b2):
    src16 = edge_index[0].reshape(NS, NI2, CK)
    dst16 = edge_index[1].reshape(NS, NI2, CK)
    ei_v = edge_index.reshape(2, ECH, 128).transpose(1, 0, 2)

    degp = _deg_kernel(ei_v)

    dinv, g1 = _tca(degp, x, W1)

    s1 = _scatter_kernel(src16, dst16, g1.reshape(2 * NP, DH))

    a1 = gamma * lax.rsqrt(run_var + EPS)
    c1 = beta - run_mean * a1
    g2 = _tcb(s1, g1, dinv, b1[None, :], a1[None, :], c1[None, :], W2)

    s2 = _scatter_kernel(src16, dst16, g2.reshape(2 * NP, DH))
    return _tcc(s2, g2, dinv, b2[None, :])


# speedup vs baseline: 41.6867x; 1.0094x over previous
import functools

import jax
import jax.numpy as jnp
from jax import lax
from jax.experimental import pallas as pl
from jax.experimental.pallas import tpu as pltpu
from jax.experimental.pallas import tpu_sc as plsc

N = 10000
E = 320000
D = 128
DH = D // 2
EPS = 1e-5

NC = 2
NS = 16
NW = NC * NS
CK = 80
ECH = E // 128
DCH = ECH // NW
DXT = ECH - DCH * NW
EPT = E // NS
NI2 = EPT // CK
NP = 10240
RPT = NP // NS
ZR = 160
RB = 2048
NBUF = 5
DGRP = 25

_mesh = plsc.VectorSubcoreMesh(core_axis_name="c", subcore_axis_name="s")


@functools.partial(
    pl.kernel,
    mesh=_mesh,
    out_type=jax.ShapeDtypeStruct((NC, NP, D), jnp.float32),
    scratch_types=[
        pltpu.VMEM((DCH + 1, 128), jnp.int32),
        pltpu.VMEM((128, 16), jnp.float32),
        pltpu.VMEM((RPT, 16), jnp.float32),
        pltpu.VMEM_SHARED((NP, 16), jnp.float32),
        pltpu.SemaphoreType.DMA,
    ],
    compiler_params=pltpu.CompilerParams(use_tc_tiling_on_sc=False),
)
def _deg_kernel(ei_hbm, out_hbm, idx_v, ones_v, buf_v, acc_sh, sem):
    c = lax.axis_index("c")
    s = lax.axis_index("s")
    wid = s * NC + c

    def fill(i, _):
        ones_v[i] = jnp.full((16,), 1.0, jnp.float32)
        return 0

    lax.fori_loop(0, 128, fill, 0)

    def fill2(i, _):
        buf_v[i] = jnp.zeros((16,), jnp.float32)
        return 0

    lax.fori_loop(0, RPT, fill2, 0)

    pltpu.sync_copy(buf_v, acc_sh.at[pl.ds(s * RPT, RPT)])
    plsc.subcore_barrier()

    pltpu.sync_copy(ei_hbm.at[pl.ds(wid * DCH, DCH), 1],
                    idx_v.at[pl.ds(0, DCH)])
    extra = wid < DXT

    @pl.when(extra)
    def _():
        pltpu.sync_copy(ei_hbm.at[NW * DCH + wid, 1], idx_v.at[DCH])

    nch = jnp.where(extra, DCH + 1, DCH)

    def fire(j, _):
        pltpu.async_copy(ones_v, acc_sh.at[idx_v.at[j]], sem, add=True)
        return 0

    def drain(j, _):
        pltpu.make_async_copy(ones_v, acc_sh.at[idx_v.at[j]], sem).wait()
        return 0

    def dgrp(g, _):
        lax.fori_loop(g * DGRP, (g + 1) * DGRP, fire, 0)
        lax.fori_loop(g * DGRP, (g + 1) * DGRP, drain, 0)
        return 0

    ngrp = DCH // DGRP
    lax.fori_loop(0, ngrp, dgrp, 0)
    lax.fori_loop(ngrp * DGRP, nch, fire, 0)
    lax.fori_loop(ngrp * DGRP, nch, drain, 0)
    plsc.subcore_barrier()

    pltpu.sync_copy(acc_sh.at[pl.ds(s * RPT, RPT)], buf_v)
    pltpu.sync_copy(buf_v, out_hbm.at[c, pl.ds(s * RPT, RPT), pl.ds(0, 16)])


@functools.partial(
    pl.kernel,
    mesh=_mesh,
    out_type=jax.ShapeDtypeStruct((NP, D), jnp.float32),
    scratch_types=[
        pltpu.VMEM((NI2, CK), jnp.int32),
        pltpu.VMEM((NI2, CK), jnp.int32),
        pltpu.VMEM((NBUF, CK, DH), jnp.float32),
        pltpu.VMEM((ZR, DH), jnp.float32),
        pltpu.VMEM_SHARED((NP, DH), jnp.float32),
    ] + [pltpu.SemaphoreType.DMA] * (2 * NBUF),
    compiler_params=pltpu.CompilerParams(use_tc_tiling_on_sc=False),
)
def _scatter_kernel(src_hbm, dst_hbm, g_hbm, out_hbm, srcv, dstv, rows_v,
                    buf_v, acc_sh, *sems):
    c = lax.axis_index("c")
    s = lax.axis_index("s")

    pltpu.sync_copy(src_hbm.at[s], srcv)
    pltpu.sync_copy(dst_hbm.at[s], dstv)

    def adjust(j, _):
        for k in range(CK // 16):
            sl = pl.ds(16 * k, 16)
            srcv[j, sl] = srcv[j, sl] * 2 + c
        return 0

    lax.fori_loop(0, NI2, adjust, 0)

    gsems = sems[:NBUF]
    ssems = sems[NBUF:]

    for b in range(NBUF):
        pltpu.async_copy(g_hbm.at[srcv.at[b]], rows_v.at[b], gsems[b])

    def fill(i, _):
        for k in range(DH // 16):
            buf_v[i, pl.ds(16 * k, 16)] = jnp.zeros((16,), jnp.float32)
        return 0

    lax.fori_loop(0, ZR, fill, 0)

    for t in range(RPT // ZR):
        pltpu.sync_copy(buf_v, acc_sh.at[pl.ds(s * RPT + t * ZR, ZR)])
    plsc.subcore_barrier()


    def grp(g, _):
        for b in range(NBUF):
            j = g * NBUF + b
            bp = (b - 1) % NBUF
            pltpu.make_async_copy(
                g_hbm.at[srcv.at[j]], rows_v.at[b], gsems[b]).wait()
            pltpu.async_copy(rows_v.at[b], acc_sh.at[dstv.at[j]], ssems[b],
                             add=True)
            pj = j + NBUF - 1

            def reissue():
                pltpu.make_async_copy(
                    rows_v.at[bp], acc_sh.at[dstv.at[j - 1]],
                    ssems[bp]).wait()
                pltpu.async_copy(g_hbm.at[srcv.at[pj]], rows_v.at[bp],
                                 gsems[bp])

            if b == 0:
                pl.when(jnp.logical_and(j >= 1, pj < NI2))(reissue)
            else:
                pl.when(pj < NI2)(reissue)
        return 0

    lax.fori_loop(0, NI2 // NBUF, grp, 0)

    last = NI2 - NBUF
    for b in range(NBUF):
        pltpu.make_async_copy(
            rows_v.at[b], acc_sh.at[dstv.at[last + b]], ssems[b]).wait()
    plsc.subcore_barrier()

    for t in range(RPT // ZR):
        sl = pl.ds(s * RPT + t * ZR, ZR)
        pltpu.sync_copy(acc_sh.at[sl], buf_v)
        pltpu.sync_copy(buf_v, out_hbm.at[sl, pl.ds(c * DH, DH)])


def _tca_body(degp_ref, x_ref, w_ref, dinv_ref, g_ref):
    deg = degp_ref[0, :, 0:1] + degp_ref[1, :, 0:1] + 1.0
    dinv = lax.rsqrt(deg)
    h = jnp.dot(x_ref[...], w_ref[...], preferred_element_type=jnp.float32)
    dinv_ref[...] = dinv
    g_ref[...] = h * dinv


_tca = pl.pallas_call(
    _tca_body,
    grid=(NP // RB,),
    in_specs=[
        pl.BlockSpec((2, RB, D), lambda i: (0, i, 0)),
        pl.BlockSpec((RB, D), lambda i: (i, 0)),
        pl.BlockSpec((D, D), lambda i: (0, 0)),
    ],
    out_specs=[
        pl.BlockSpec((RB, 1), lambda i: (i, 0)),
        pl.BlockSpec((RB, D), lambda i: (i, 0)),
    ],
    out_shape=[
        jax.ShapeDtypeStruct((NP, 1), jnp.float32),
        jax.ShapeDtypeStruct((NP, D), jnp.float32),
    ],
)


def _tcb_body(sp_ref, g1_ref, dinv_ref, b1_ref, a1_ref, c1_ref, w2_ref,
              g2_ref):
    sval = sp_ref[...] + g1_ref[...]
    dinv = dinv_ref[...]
    o = jnp.maximum(sval * dinv + b1_ref[...], 0.0)
    h = o * a1_ref[...] + c1_ref[...]
    g2_ref[...] = jnp.dot(h, w2_ref[...],
                          preferred_element_type=jnp.float32) * dinv


_tcb = pl.pallas_call(
    _tcb_body,
    grid=(NP // RB,),
    in_specs=[
        pl.BlockSpec((RB, D), lambda i: (i, 0)),
        pl.BlockSpec((RB, D), lambda i: (i, 0)),
        pl.BlockSpec((RB, 1), lambda i: (i, 0)),
        pl.BlockSpec((1, D), lambda i: (0, 0)),
        pl.BlockSpec((1, D), lambda i: (0, 0)),
        pl.BlockSpec((1, D), lambda i: (0, 0)),
        pl.BlockSpec((D, D), lambda i: (0, 0)),
    ],
    out_specs=pl.BlockSpec((RB, D), lambda i: (i, 0)),
    out_shape=jax.ShapeDtypeStruct((NP, D), jnp.float32),
)


def _tcc_body(sp_ref, g2_ref, dinv_ref, b2_ref, y_ref):
    sval = sp_ref[...] + g2_ref[...]
    y_ref[...] = jnp.maximum(sval * dinv_ref[...] + b2_ref[...], 0.0)


_tcc = pl.pallas_call(
    _tcc_body,
    grid=(NP // RB,),
    in_specs=[
        pl.BlockSpec((RB, D), lambda i: (i, 0)),
        pl.BlockSpec((RB, D), lambda i: (i, 0)),
        pl.BlockSpec((RB, 1), lambda i: (i, 0)),
        pl.BlockSpec((1, D), lambda i: (0, 0)),
    ],
    out_specs=pl.BlockSpec((RB, D), lambda i: (i, 0)),
    out_shape=jax.ShapeDtypeStruct((N, D), jnp.float32),
)


def kernel(x, edge_index, batch, W1, b1, gamma, beta, run_mean, run_var,
           W2, b2):
    src16 = edge_index[0].reshape(NS, NI2, CK)
    dst16 = edge_index[1].reshape(NS, NI2, CK)
    ei_v = edge_index.reshape(2, ECH, 128).transpose(1, 0, 2)

    degp = _deg_kernel(ei_v)

    dinv, g1 = _tca(degp, x, W1)

    s1 = _scatter_kernel(src16, dst16, g1.reshape(2 * NP, DH))

    a1 = gamma * lax.rsqrt(run_var + EPS)
    c1 = beta - run_mean * a1
    g2 = _tcb(s1, g1, dinv, b1[None, :], a1[None, :], c1[None, :], W2)

    s2 = _scatter_kernel(src16, dst16, g2.reshape(2 * NP, DH))
    return _tcc(s2, g2, dinv, b2[None, :])
